# baseline (speedup 1.0000x reference)
import functools

import jax
import jax.numpy as jnp
from jax import lax
from jax.experimental import pallas as pl
from jax.experimental.pallas import tpu as pltpu
from jax.experimental.pallas import tpu_sc as plsc

_TAU = 0.07
_B = 8
_N = 16384
_D = 64
_P = 3
_S = 512
_NQ = 4
_SL = 128


def _monotonic_i32(u):
    b = lax.bitcast_convert_type(u, jnp.int32)
    return b ^ ((b >> 31) & jnp.int32(0x7FFFFFFF))


def _select_kernel(sn_ref, pred_ref, mask_ref, idx_out_ref, cnt_out_ref):
    f32 = jnp.float32

    def _f(x):
        return x * jnp.log(x + 1e-6)

    u = (_f(pred_ref[:, 0, :]) + _f(pred_ref[:, 1, :])
         + _f(pred_ref[:, 2, :]) + _f(pred_ref[:, 3, :]))
    s = _monotonic_i32(u)
    m = mask_ref[...]
    hardm = m == 0
    easym = jnp.logical_not(hardm)
    nh = jnp.sum(hardm.astype(jnp.int32), axis=1, keepdims=True)
    ne = _N - nh
    sn = sn_ref[0]
    hsn = jnp.minimum(sn // 2, nh)
    esn = jnp.minimum(sn - hsn, ne)

    imin = jnp.iinfo(jnp.int32).min
    imax = jnp.iinfo(jnp.int32).max
    lo0 = jnp.full((_B, 1), imin, jnp.int32)
    hi0 = jnp.full((_B, 1), imax, jnp.int32)

    def bs_body(_, c):
        lo_h, hi_h, lo_e, hi_e = c
        mid_h = (lo_h >> 1) + (hi_h >> 1) + (lo_h & hi_h & 1)
        mid_e = (lo_e >> 1) + (hi_e >> 1) + (lo_e & hi_e & 1)
        mid = jnp.where(hardm, mid_h, mid_e)
        ge = s >= mid
        cnt_h = jnp.sum((hardm & ge).astype(jnp.int32), axis=1, keepdims=True)
        cnt_e = jnp.sum((easym & ge).astype(jnp.int32), axis=1, keepdims=True)
        ph = cnt_h >= hsn
        pe = cnt_e >= esn
        return (jnp.where(ph, mid_h, lo_h), jnp.where(ph, hi_h, mid_h),
                jnp.where(pe, mid_e, lo_e), jnp.where(pe, hi_e, mid_e))

    t_h, _, t_e, _ = lax.fori_loop(0, 32, bs_body, (lo0, hi0, lo0, hi0))

    need_h = (hsn - jnp.sum((hardm & (s > t_h)).astype(jnp.int32), axis=1,
                            keepdims=True)).astype(f32)
    need_e = (esn - jnp.sum((easym & (s > t_e)).astype(jnp.int32), axis=1,
                            keepdims=True)).astype(f32)

    ri = lax.broadcasted_iota(jnp.int32, (128, 128), 0)
    ci = lax.broadcasted_iota(jnp.int32, (128, 128), 1)
    UT = (ri <= ci).astype(f32)
    LT = (ci < ri).astype(f32)
    ONES = jnp.ones((128, 128), f32)
    dn = (((0,), (0,)), ((), ()))
    dnn = (((1,), (0,)), ((), ()))

    def mm(a, b):
        return lax.dot_general(a, b, dnn, preferred_element_type=f32)

    def excl_prefix(x):
        incl = mm(x, UT)
        return (incl - x) + mm(LT, mm(x, ONES))

    cnts = []
    EYE = (ri == ci).astype(f32)
    dT = (((0,), (0,)), ((), ()))
    d32 = (((2,), (0,)), ((), ()))
    iota_r3 = lax.broadcasted_iota(jnp.int32, (1, 1, 128), 2).astype(f32)
    iota_c3 = iota_r3
    sval2 = (lax.broadcasted_iota(jnp.int32, (_NQ, _SL), 0) * _SL
             + lax.broadcasted_iota(jnp.int32, (_NQ, _SL), 1)).astype(f32)

    for b in range(_B):
        s_b = s[b].reshape(128, 128)
        hard_b = m[b].reshape(128, 128) == 0
        easy_b = jnp.logical_not(hard_b)
        t_hb = lax.slice(t_h, (b, 0), (b + 1, 1))
        t_eb = lax.slice(t_e, (b, 0), (b + 1, 1))
        need_hb = lax.slice(need_h, (b, 0), (b + 1, 1))
        need_eb = lax.slice(need_e, (b, 0), (b + 1, 1))

        tie_h = (hard_b & (s_b == t_hb)).astype(f32)
        tie_e = (easy_b & (s_b == t_eb)).astype(f32)
        rank_h = excl_prefix(tie_h)
        rank_e = excl_prefix(tie_e)
        sel_b = ((hard_b & ((s_b > t_hb) |
                            ((s_b == t_hb) & (rank_h < need_hb)))) |
                 (easy_b & ((s_b > t_eb) |
                            ((s_b == t_eb) & (rank_e < need_eb)))))
        self_f = sel_b.astype(f32)
        cnts.append(jnp.sum(self_f.astype(jnp.int32), keepdims=True)
                    .reshape(1, 1))
        posin = mm(self_f, UT)
        rbc = mm(LT, mm(self_f, ONES))
        roffT = lax.dot_general(rbc, EYE, dT,
                                preferred_element_type=f32)
        roff_lanes = lax.slice(roffT, (0, 0), (1, 128)).reshape(1, 1, 128)
        r_s = (jnp.sum((roff_lanes <= sval2[:, :, None]).astype(f32), axis=2)
               - 1.0)
        onehot3 = (r_s[:, :, None] == iota_r3).astype(f32)
        rg = lax.dot_general(onehot3, posin, d32,
                             preferred_element_type=f32)
        sg = lax.dot_general(onehot3, self_f, d32,
                             preferred_element_type=f32)
        ro = lax.dot_general(onehot3, rbc, d32,
                             preferred_element_type=f32)
        kk = sval2[:, :, None] - ro + 1.0
        hit = ((rg == kk).astype(f32) * sg)
        c_s = jnp.sum(hit * iota_c3, axis=2)
        idx_b = (128.0 * r_s + c_s).astype(jnp.int32)
        idx_out_ref[b] = idx_b

    cnt_out_ref[...] = jnp.concatenate(cnts, axis=0).astype(jnp.int32)


def _run_select(pred_t, mask2, sn):
    return pl.pallas_call(
        _select_kernel,
        in_specs=[
            pl.BlockSpec(memory_space=pltpu.SMEM),
            pl.BlockSpec(memory_space=pltpu.VMEM),
            pl.BlockSpec(memory_space=pltpu.VMEM),
        ],
        out_specs=[
            pl.BlockSpec(memory_space=pltpu.VMEM),
            pl.BlockSpec(memory_space=pltpu.VMEM),
        ],
        out_shape=[
            jax.ShapeDtypeStruct((_B, _NQ, _SL), jnp.int32),
            jax.ShapeDtypeStruct((_B, 1), jnp.int32),
        ],
    )(sn, pred_t, mask2)


_NROWS = _P * _D
_NSTREAM = 8
_CHUNK = _NROWS * _SL // _NSTREAM


def _gather_body(proj_hbm, idx_hbm, out_hbm, idx_v, gidx, buf, sem):
    c = lax.axis_index("c")
    sub = lax.axis_index("s")
    wid = sub * 2 + c
    b = wid // _NQ
    q = lax.rem(wid, _NQ)
    pltpu.sync_copy(idx_hbm.at[b, q], idx_v)

    boff = b * (_D * _N)

    def build(rid, carry):
        i = rid // _D
        dd = lax.rem(rid, _D)
        base = (i * (_B * _D) + dd) * _N + boff
        for k in range(_SL // 16):
            gidx[pl.ds(rid * _SL + k * 16, 16)] = (
                idx_v[pl.ds(k * 16, 16)] + base)
        return carry

    lax.fori_loop(0, _NROWS, build, 0)

    copies = []
    for s in range(_NSTREAM):
        cp = pltpu.make_async_copy(
            proj_hbm.at[gidx.at[pl.ds(s * _CHUNK, _CHUNK)]],
            buf.at[pl.ds(s * _CHUNK, _CHUNK)], sem)
        cp.start()
        copies.append(cp)
    for cp in copies:
        cp.wait()

    pltpu.sync_copy(buf, out_hbm.at[b, q])


def _run_gather(proj4, idx_sel):
    mesh = plsc.VectorSubcoreMesh(core_axis_name="c", subcore_axis_name="s")
    fn = pl.kernel(
        _gather_body,
        out_type=jax.ShapeDtypeStruct((_B, _NQ, _P * _D * _SL), jnp.float32),
        mesh=mesh,
        compiler_params=pltpu.CompilerParams(
            needs_layout_passes=False, use_tc_tiling_on_sc=False),
        scratch_types=[
            pltpu.VMEM((_SL,), jnp.int32),
            pltpu.VMEM((_NROWS * _SL,), jnp.int32),
            pltpu.VMEM((_NROWS * _SL,), jnp.float32),
            pltpu.SemaphoreType.DMA,
        ],
    )
    return fn(proj4, idx_sel)


def _loss_kernel(idxp_ref, g_ref, cnt_ref, out_ref):
    f32 = jnp.float32
    oh = [(idxp_ref[0] == p).astype(f32) for p in range(_P)]
    ii = lax.broadcasted_iota(jnp.int32, (_S, _S), 0)
    jj = lax.broadcasted_iota(jnp.int32, (_S, _S), 1)
    eye = (ii == jj).astype(f32)
    iota_row = lax.broadcasted_iota(jnp.int32, (1, _S), 1)
    dnT = (((0,), (0,)), ((), ()))
    acc = jnp.zeros((1, 1), f32)
    for b in range(_B):
        cnt_b = lax.slice(cnt_ref[...], (b, 0), (b + 1, 1))
        vs = []
        for p in range(_P):
            blocks = [g_ref[b, q, p] for q in range(_NQ)]
            cmat = jnp.concatenate(blocks, axis=1)
            n1 = jnp.sqrt(jnp.sum(cmat * cmat, axis=0, keepdims=True))
            v = cmat / jnp.maximum(n1, 1e-12)
            n2 = jnp.sqrt(jnp.sum(v * v, axis=0, keepdims=True))
            vs.append(v / jnp.maximum(n2, 1e-8))
        cur = oh[0] * vs[0] + oh[1] * vs[1] + oh[2] * vs[2]
        possum = (sum(jnp.sum(cur * vs[p], axis=0, keepdims=True)
                      for p in range(_P))
                  - jnp.sum(cur * cur, axis=0, keepdims=True))
        pos_item = jnp.exp(possum / _TAU)
        mat = lax.dot_general(cur, cur, dnT, preferred_element_type=f32)
        me = jnp.exp(mat / _TAU)
        valid_i = (ii < cnt_b).astype(f32)
        negsum = jnp.sum(me * valid_i, axis=0, keepdims=True)
        diag = jnp.sum(me * eye, axis=0, keepdims=True)
        neg = negsum - diag
        per = -jnp.log(pos_item / (pos_item + neg + 1e-8))
        validj = (iota_row < cnt_b).astype(f32)
        acc = acc + (jnp.sum(per * validj, keepdims=True).reshape(1, 1)
                     / cnt_b.astype(f32))
    out_ref[...] = acc / float(_B)


def _run_loss(idxp, gathered, cnt):
    return pl.pallas_call(
        _loss_kernel,
        in_specs=[
            pl.BlockSpec(memory_space=pltpu.SMEM),
            pl.BlockSpec(memory_space=pltpu.VMEM),
            pl.BlockSpec(memory_space=pltpu.VMEM),
        ],
        out_specs=pl.BlockSpec(memory_space=pltpu.VMEM),
        out_shape=jax.ShapeDtypeStruct((1, 1), jnp.float32),
    )(idxp, gathered.reshape(_B, _NQ, _P, _D, _SL), cnt)


def kernel(pred, proj_list, idx, pseudo_label, mask, sample_num):
    del pseudo_label
    pred_t = pred.reshape(_B, 4, _N)
    mask2 = mask.reshape(_B, _N).astype(jnp.int32)
    sn = jnp.asarray(sample_num, jnp.int32).reshape(1)
    idxp = jnp.asarray(idx, jnp.int32).reshape(1)
    idx_sel, cnt = _run_select(pred_t, mask2, sn)
    proj4 = proj_list.reshape(_P * _B * _D * _N)
    gathered = _run_gather(proj4, idx_sel)
    out = _run_loss(idxp, gathered, cnt)
    return out.reshape(())


# speedup vs baseline: 1.3214x; 1.3214x over previous
import functools

import jax
import jax.numpy as jnp
from jax import lax
from jax.experimental import pallas as pl
from jax.experimental.pallas import tpu as pltpu
from jax.experimental.pallas import tpu_sc as plsc

_TAU = 0.07
_B = 8
_N = 16384
_D = 64
_P = 3
_S = 512
_NQ = 4
_SL = 128


def _monotonic_i32(u):
    b = lax.bitcast_convert_type(u, jnp.int32)
    return b ^ ((b >> 31) & jnp.int32(0x7FFFFFFF))


def _select_kernel(sn_ref, pred_ref, mask_ref, idx_out_ref, cnt_out_ref):
    f32 = jnp.float32

    def _f(x):
        return x * jnp.log(x + 1e-6)

    u = _f(pred_ref[0]) + _f(pred_ref[1]) + _f(pred_ref[2]) + _f(pred_ref[3])
    s = _monotonic_i32(u)
    m = mask_ref[...]
    hardm = m == 0
    easym = jnp.logical_not(hardm)
    nh = jnp.sum(hardm.astype(jnp.int32), axis=1, keepdims=True)
    ne = _N - nh
    sn = sn_ref[0]
    hsn = jnp.minimum(sn // 2, nh)
    esn = jnp.minimum(sn - hsn, ne)

    imin = jnp.iinfo(jnp.int32).min
    imax = jnp.iinfo(jnp.int32).max
    lo0 = jnp.full((_B, 1), imin, jnp.int32)
    hi0 = jnp.full((_B, 1), imax, jnp.int32)

    def bs_body(_, c):
        lo_h, hi_h, lo_e, hi_e = c
        mid_h = (lo_h >> 1) + (hi_h >> 1) + (lo_h & hi_h & 1)
        mid_e = (lo_e >> 1) + (hi_e >> 1) + (lo_e & hi_e & 1)
        mid = jnp.where(hardm, mid_h, mid_e)
        ge = s >= mid
        cnt_h = jnp.sum((hardm & ge).astype(jnp.int32), axis=1, keepdims=True)
        cnt_e = jnp.sum((easym & ge).astype(jnp.int32), axis=1, keepdims=True)
        ph = cnt_h >= hsn
        pe = cnt_e >= esn
        return (jnp.where(ph, mid_h, lo_h), jnp.where(ph, hi_h, mid_h),
                jnp.where(pe, mid_e, lo_e), jnp.where(pe, hi_e, mid_e))

    t_h, _, t_e, _ = lax.fori_loop(0, 32, bs_body, (lo0, hi0, lo0, hi0))

    need_h = (hsn - jnp.sum((hardm & (s > t_h)).astype(jnp.int32), axis=1,
                            keepdims=True)).astype(f32)
    need_e = (esn - jnp.sum((easym & (s > t_e)).astype(jnp.int32), axis=1,
                            keepdims=True)).astype(f32)

    ri = lax.broadcasted_iota(jnp.int32, (128, 128), 0)
    ci = lax.broadcasted_iota(jnp.int32, (128, 128), 1)
    UT = (ri <= ci).astype(f32)
    LT = (ci < ri).astype(f32)
    ONES = jnp.ones((128, 128), f32)
    dn = (((0,), (0,)), ((), ()))
    dnn = (((1,), (0,)), ((), ()))

    def mm(a, b):
        return lax.dot_general(a, b, dnn, preferred_element_type=f32)

    def excl_prefix(x):
        incl = mm(x, UT)
        return (incl - x) + mm(LT, mm(x, ONES))

    cnts = []
    EYE = (ri == ci).astype(f32)
    dT = (((0,), (0,)), ((), ()))
    d32 = (((2,), (0,)), ((), ()))
    iota_r3 = lax.broadcasted_iota(jnp.int32, (1, 1, 128), 2).astype(f32)
    iota_c3 = iota_r3
    sval2 = (lax.broadcasted_iota(jnp.int32, (_NQ, _SL), 0) * _SL
             + lax.broadcasted_iota(jnp.int32, (_NQ, _SL), 1)).astype(f32)

    for b in range(_B):
        s_b = s[b].reshape(128, 128)
        hard_b = m[b].reshape(128, 128) == 0
        easy_b = jnp.logical_not(hard_b)
        t_hb = lax.slice(t_h, (b, 0), (b + 1, 1))
        t_eb = lax.slice(t_e, (b, 0), (b + 1, 1))
        need_hb = lax.slice(need_h, (b, 0), (b + 1, 1))
        need_eb = lax.slice(need_e, (b, 0), (b + 1, 1))

        tie_h = (hard_b & (s_b == t_hb)).astype(f32)
        tie_e = (easy_b & (s_b == t_eb)).astype(f32)
        rank_h = excl_prefix(tie_h)
        rank_e = excl_prefix(tie_e)
        sel_b = ((hard_b & ((s_b > t_hb) |
                            ((s_b == t_hb) & (rank_h < need_hb)))) |
                 (easy_b & ((s_b > t_eb) |
                            ((s_b == t_eb) & (rank_e < need_eb)))))
        self_f = sel_b.astype(f32)
        cnts.append(jnp.sum(self_f.astype(jnp.int32), keepdims=True)
                    .reshape(1, 1))
        posin = mm(self_f, UT)
        rbc = mm(LT, mm(self_f, ONES))
        roffT = lax.dot_general(rbc, EYE, dT,
                                preferred_element_type=f32)
        roff_lanes = lax.slice(roffT, (0, 0), (1, 128)).reshape(1, 1, 128)
        r_s = (jnp.sum((roff_lanes <= sval2[:, :, None]).astype(f32), axis=2)
               - 1.0)
        onehot3 = (r_s[:, :, None] == iota_r3).astype(f32)
        rg = lax.dot_general(onehot3, posin, d32,
                             preferred_element_type=f32)
        sg = lax.dot_general(onehot3, self_f, d32,
                             preferred_element_type=f32)
        ro = lax.dot_general(onehot3, rbc, d32,
                             preferred_element_type=f32)
        kk = sval2[:, :, None] - ro + 1.0
        hit = ((rg == kk).astype(f32) * sg)
        c_s = jnp.sum(hit * iota_c3, axis=2)
        idx_b = (128.0 * r_s + c_s).astype(jnp.int32)
        idx_out_ref[b] = idx_b

    cnt_out_ref[...] = jnp.concatenate(cnts, axis=0).astype(jnp.int32)


def _run_select(pred_t, mask2, sn):
    return pl.pallas_call(
        _select_kernel,
        in_specs=[
            pl.BlockSpec(memory_space=pltpu.SMEM),
            pl.BlockSpec(memory_space=pltpu.VMEM),
            pl.BlockSpec(memory_space=pltpu.VMEM),
        ],
        out_specs=[
            pl.BlockSpec(memory_space=pltpu.VMEM),
            pl.BlockSpec(memory_space=pltpu.VMEM),
        ],
        out_shape=[
            jax.ShapeDtypeStruct((_B, _NQ, _SL), jnp.int32),
            jax.ShapeDtypeStruct((_B, 1), jnp.int32),
        ],
    )(sn, pred_t, mask2)


_NROWS = _P * _D
_NSTREAM = 8
_CHUNK = _NROWS * _SL // _NSTREAM


def _gather_body(proj_hbm, idx_hbm, out_hbm, idx_v, gidx, buf, sem):
    c = lax.axis_index("c")
    sub = lax.axis_index("s")
    wid = sub * 2 + c
    b = wid // _NQ
    q = lax.rem(wid, _NQ)
    pltpu.sync_copy(idx_hbm.at[b, q], idx_v)

    boff = b * (_D * _N)

    def build(rid, carry):
        i = rid // _D
        dd = lax.rem(rid, _D)
        base = (i * (_B * _D) + dd) * _N + boff
        for k in range(_SL // 16):
            gidx[pl.ds(rid * _SL + k * 16, 16)] = (
                idx_v[pl.ds(k * 16, 16)] + base)
        return carry

    lax.fori_loop(0, _NROWS, build, 0)

    copies = []
    for s in range(_NSTREAM):
        cp = pltpu.make_async_copy(
            proj_hbm.at[gidx.at[pl.ds(s * _CHUNK, _CHUNK)]],
            buf.at[pl.ds(s * _CHUNK, _CHUNK)], sem)
        cp.start()
        copies.append(cp)
    for cp in copies:
        cp.wait()

    pltpu.sync_copy(buf, out_hbm.at[b, q])


def _run_gather(proj4, idx_sel):
    mesh = plsc.VectorSubcoreMesh(core_axis_name="c", subcore_axis_name="s")
    fn = pl.kernel(
        _gather_body,
        out_type=jax.ShapeDtypeStruct((_B, _NQ, _P * _D * _SL), jnp.float32),
        mesh=mesh,
        compiler_params=pltpu.CompilerParams(
            needs_layout_passes=False, use_tc_tiling_on_sc=False),
        scratch_types=[
            pltpu.VMEM((_SL,), jnp.int32),
            pltpu.VMEM((_NROWS * _SL,), jnp.int32),
            pltpu.VMEM((_NROWS * _SL,), jnp.float32),
            pltpu.SemaphoreType.DMA,
        ],
    )
    return fn(proj4, idx_sel)


def _loss_kernel(idxp_ref, g_ref, cnt_ref, out_ref):
    f32 = jnp.float32
    oh = [(idxp_ref[0] == p).astype(f32) for p in range(_P)]
    ii = lax.broadcasted_iota(jnp.int32, (_S, _S), 0)
    jj = lax.broadcasted_iota(jnp.int32, (_S, _S), 1)
    eye = (ii == jj).astype(f32)
    iota_row = lax.broadcasted_iota(jnp.int32, (1, _S), 1)
    dnT = (((0,), (0,)), ((), ()))
    acc = jnp.zeros((1, 1), f32)
    for b in range(_B):
        cnt_b = lax.slice(cnt_ref[...], (b, 0), (b + 1, 1))
        vs = []
        for p in range(_P):
            blocks = [g_ref[b, q, p] for q in range(_NQ)]
            cmat = jnp.concatenate(blocks, axis=1)
            n1 = jnp.sqrt(jnp.sum(cmat * cmat, axis=0, keepdims=True))
            v = cmat / jnp.maximum(n1, 1e-12)
            n2 = jnp.sqrt(jnp.sum(v * v, axis=0, keepdims=True))
            vs.append(v / jnp.maximum(n2, 1e-8))
        cur = oh[0] * vs[0] + oh[1] * vs[1] + oh[2] * vs[2]
        possum = (sum(jnp.sum(cur * vs[p], axis=0, keepdims=True)
                      for p in range(_P))
                  - jnp.sum(cur * cur, axis=0, keepdims=True))
        pos_item = jnp.exp(possum / _TAU)
        mat = lax.dot_general(cur, cur, dnT, preferred_element_type=f32)
        me = jnp.exp(mat / _TAU)
        valid_i = (ii < cnt_b).astype(f32)
        negsum = jnp.sum(me * valid_i, axis=0, keepdims=True)
        diag = jnp.sum(me * eye, axis=0, keepdims=True)
        neg = negsum - diag
        per = -jnp.log(pos_item / (pos_item + neg + 1e-8))
        validj = (iota_row < cnt_b).astype(f32)
        acc = acc + (jnp.sum(per * validj, keepdims=True).reshape(1, 1)
                     / cnt_b.astype(f32))
    out_ref[...] = acc / float(_B)


def _run_loss(idxp, gathered, cnt):
    return pl.pallas_call(
        _loss_kernel,
        in_specs=[
            pl.BlockSpec(memory_space=pltpu.SMEM),
            pl.BlockSpec(memory_space=pltpu.VMEM),
            pl.BlockSpec(memory_space=pltpu.VMEM),
        ],
        out_specs=pl.BlockSpec(memory_space=pltpu.VMEM),
        out_shape=jax.ShapeDtypeStruct((1, 1), jnp.float32),
    )(idxp, gathered.reshape(_B, _NQ, _P, _D, _SL), cnt)


def kernel(pred, proj_list, idx, pseudo_label, mask, sample_num):
    del pseudo_label
    pred_t = jnp.transpose(pred.reshape(_B, 4, _N), (1, 0, 2))
    mask2 = mask.reshape(_B, _N).astype(jnp.int32)
    sn = jnp.asarray(sample_num, jnp.int32).reshape(1)
    idxp = jnp.asarray(idx, jnp.int32).reshape(1)
    idx_sel, cnt = _run_select(pred_t, mask2, sn)
    proj4 = proj_list.reshape(_P * _B * _D * _N)
    gathered = _run_gather(proj4, idx_sel)
    out = _run_loss(idxp, gathered, cnt)
    return out.reshape(())


# speedup vs baseline: 1.4068x; 1.0646x over previous
import functools

import jax
import jax.numpy as jnp
from jax import lax
from jax.experimental import pallas as pl
from jax.experimental.pallas import tpu as pltpu
from jax.experimental.pallas import tpu_sc as plsc

_TAU = 0.07
_B = 8
_N = 16384
_D = 64
_P = 3
_S = 448
_NQ = 4
_SL = 112


def _monotonic_i32(u):
    b = lax.bitcast_convert_type(u, jnp.int32)
    return b ^ ((b >> 31) & jnp.int32(0x7FFFFFFF))


def _select_kernel(sn_ref, pred_ref, mask_ref, idx_out_ref, cnt_out_ref):
    f32 = jnp.float32

    def _f(x):
        return x * jnp.log(x + 1e-6)

    u = _f(pred_ref[0]) + _f(pred_ref[1]) + _f(pred_ref[2]) + _f(pred_ref[3])
    s = _monotonic_i32(u)
    m = mask_ref[...]
    hardm = m == 0
    easym = jnp.logical_not(hardm)
    nh = jnp.sum(hardm.astype(jnp.int32), axis=1, keepdims=True)
    ne = _N - nh
    sn = sn_ref[0]
    hsn = jnp.minimum(sn // 2, nh)
    esn = jnp.minimum(sn - hsn, ne)

    imin = jnp.iinfo(jnp.int32).min
    imax = jnp.iinfo(jnp.int32).max
    lo0 = jnp.full((_B, 1), imin, jnp.int32)
    hi0 = jnp.full((_B, 1), imax, jnp.int32)

    def bs_body(_, c):
        lo_h, hi_h, lo_e, hi_e = c
        mid_h = (lo_h >> 1) + (hi_h >> 1) + (lo_h & hi_h & 1)
        mid_e = (lo_e >> 1) + (hi_e >> 1) + (lo_e & hi_e & 1)
        mid = jnp.where(hardm, mid_h, mid_e)
        ge = s >= mid
        cnt_h = jnp.sum((hardm & ge).astype(jnp.int32), axis=1, keepdims=True)
        cnt_e = jnp.sum((easym & ge).astype(jnp.int32), axis=1, keepdims=True)
        ph = cnt_h >= hsn
        pe = cnt_e >= esn
        return (jnp.where(ph, mid_h, lo_h), jnp.where(ph, hi_h, mid_h),
                jnp.where(pe, mid_e, lo_e), jnp.where(pe, hi_e, mid_e))

    t_h, _, t_e, _ = lax.fori_loop(0, 32, bs_body, (lo0, hi0, lo0, hi0))

    need_h = (hsn - jnp.sum((hardm & (s > t_h)).astype(jnp.int32), axis=1,
                            keepdims=True)).astype(f32)
    need_e = (esn - jnp.sum((easym & (s > t_e)).astype(jnp.int32), axis=1,
                            keepdims=True)).astype(f32)

    ri = lax.broadcasted_iota(jnp.int32, (128, 128), 0)
    ci = lax.broadcasted_iota(jnp.int32, (128, 128), 1)
    UT = (ri <= ci).astype(f32)
    LT = (ci < ri).astype(f32)
    ONES = jnp.ones((128, 128), f32)
    dn = (((0,), (0,)), ((), ()))
    dnn = (((1,), (0,)), ((), ()))

    def mm(a, b):
        return lax.dot_general(a, b, dnn, preferred_element_type=f32)

    def excl_prefix(x):
        incl = mm(x, UT)
        return (incl - x) + mm(LT, mm(x, ONES))

    cnts = []
    EYE = (ri == ci).astype(f32)
    dT = (((0,), (0,)), ((), ()))
    d32 = (((2,), (0,)), ((), ()))
    iota_r3 = lax.broadcasted_iota(jnp.int32, (1, 1, 128), 2).astype(f32)
    iota_c3 = iota_r3
    sval2 = (lax.broadcasted_iota(jnp.int32, (_NQ, _SL), 0) * _SL
             + lax.broadcasted_iota(jnp.int32, (_NQ, _SL), 1)).astype(f32)

    for b in range(_B):
        s_b = s[b].reshape(128, 128)
        hard_b = m[b].reshape(128, 128) == 0
        easy_b = jnp.logical_not(hard_b)
        t_hb = lax.slice(t_h, (b, 0), (b + 1, 1))
        t_eb = lax.slice(t_e, (b, 0), (b + 1, 1))
        need_hb = lax.slice(need_h, (b, 0), (b + 1, 1))
        need_eb = lax.slice(need_e, (b, 0), (b + 1, 1))

        tie_h = (hard_b & (s_b == t_hb)).astype(f32)
        tie_e = (easy_b & (s_b == t_eb)).astype(f32)
        rank_h = excl_prefix(tie_h)
        rank_e = excl_prefix(tie_e)
        sel_b = ((hard_b & ((s_b > t_hb) |
                            ((s_b == t_hb) & (rank_h < need_hb)))) |
                 (easy_b & ((s_b > t_eb) |
                            ((s_b == t_eb) & (rank_e < need_eb)))))
        self_f = sel_b.astype(f32)
        cnts.append(jnp.sum(self_f.astype(jnp.int32), keepdims=True)
                    .reshape(1, 1))
        posin = mm(self_f, UT)
        rbc = mm(LT, mm(self_f, ONES))
        roffT = lax.dot_general(rbc, EYE, dT,
                                preferred_element_type=f32)
        roff_lanes = lax.slice(roffT, (0, 0), (1, 128)).reshape(1, 1, 128)
        r_s = (jnp.sum((roff_lanes <= sval2[:, :, None]).astype(f32), axis=2)
               - 1.0)
        onehot3 = (r_s[:, :, None] == iota_r3).astype(f32)
        rg = lax.dot_general(onehot3, posin, d32,
                             preferred_element_type=f32)
        sg = lax.dot_general(onehot3, self_f, d32,
                             preferred_element_type=f32)
        ro = lax.dot_general(onehot3, rbc, d32,
                             preferred_element_type=f32)
        kk = sval2[:, :, None] - ro + 1.0
        hit = ((rg == kk).astype(f32) * sg)
        c_s = jnp.sum(hit * iota_c3, axis=2)
        idx_b = (128.0 * r_s + c_s).astype(jnp.int32)
        idx_out_ref[b] = idx_b

    cnt_out_ref[...] = jnp.concatenate(cnts, axis=0).astype(jnp.int32)


def _run_select(pred_t, mask2, sn):
    return pl.pallas_call(
        _select_kernel,
        in_specs=[
            pl.BlockSpec(memory_space=pltpu.SMEM),
            pl.BlockSpec(memory_space=pltpu.VMEM),
            pl.BlockSpec(memory_space=pltpu.VMEM),
        ],
        out_specs=[
            pl.BlockSpec(memory_space=pltpu.VMEM),
            pl.BlockSpec(memory_space=pltpu.VMEM),
        ],
        out_shape=[
            jax.ShapeDtypeStruct((_B, _NQ, _SL), jnp.int32),
            jax.ShapeDtypeStruct((_B, 1), jnp.int32),
        ],
    )(sn, pred_t, mask2)


_NROWS = _P * _D
_NSTREAM = 8
_CHUNK = _NROWS * _SL // _NSTREAM


def _gather_body(proj_hbm, idx_hbm, out_hbm, idx_v, gidx, buf, sem):
    c = lax.axis_index("c")
    sub = lax.axis_index("s")
    wid = sub * 2 + c
    b = wid // _NQ
    q = lax.rem(wid, _NQ)
    pltpu.sync_copy(idx_hbm.at[b, q], idx_v)

    boff = b * (_D * _N)

    def build(rid, carry):
        i = rid // _D
        dd = lax.rem(rid, _D)
        base = (i * (_B * _D) + dd) * _N + boff
        for k in range(_SL // 16):
            gidx[pl.ds(rid * _SL + k * 16, 16)] = (
                idx_v[pl.ds(k * 16, 16)] + base)
        return carry

    lax.fori_loop(0, _NROWS, build, 0)

    copies = []
    for s in range(_NSTREAM):
        cp = pltpu.make_async_copy(
            proj_hbm.at[gidx.at[pl.ds(s * _CHUNK, _CHUNK)]],
            buf.at[pl.ds(s * _CHUNK, _CHUNK)], sem)
        cp.start()
        copies.append(cp)
    for cp in copies:
        cp.wait()

    pltpu.sync_copy(buf, out_hbm.at[b, q])


def _run_gather(proj4, idx_sel):
    mesh = plsc.VectorSubcoreMesh(core_axis_name="c", subcore_axis_name="s")
    fn = pl.kernel(
        _gather_body,
        out_type=jax.ShapeDtypeStruct((_B, _NQ, _P * _D * _SL), jnp.float32),
        mesh=mesh,
        compiler_params=pltpu.CompilerParams(
            needs_layout_passes=False, use_tc_tiling_on_sc=False),
        scratch_types=[
            pltpu.VMEM((_SL,), jnp.int32),
            pltpu.VMEM((_NROWS * _SL,), jnp.int32),
            pltpu.VMEM((_NROWS * _SL,), jnp.float32),
            pltpu.SemaphoreType.DMA,
        ],
    )
    return fn(proj4, idx_sel)


def _loss_kernel(idxp_ref, g_ref, cnt_ref, out_ref):
    f32 = jnp.float32
    oh = [(idxp_ref[0] == p).astype(f32) for p in range(_P)]
    ii = lax.broadcasted_iota(jnp.int32, (_S, _S), 0)
    jj = lax.broadcasted_iota(jnp.int32, (_S, _S), 1)
    eye = (ii == jj).astype(f32)
    iota_row = lax.broadcasted_iota(jnp.int32, (1, _S), 1)
    dnT = (((0,), (0,)), ((), ()))
    acc = jnp.zeros((1, 1), f32)
    for b in range(_B):
        cnt_b = lax.slice(cnt_ref[...], (b, 0), (b + 1, 1))
        vs = []
        for p in range(_P):
            blocks = [g_ref[b, q, p] for q in range(_NQ)]
            cmat = jnp.concatenate(blocks, axis=1)
            n1 = jnp.sqrt(jnp.sum(cmat * cmat, axis=0, keepdims=True))
            v = cmat / jnp.maximum(n1, 1e-12)
            n2 = jnp.sqrt(jnp.sum(v * v, axis=0, keepdims=True))
            vs.append(v / jnp.maximum(n2, 1e-8))
        cur = oh[0] * vs[0] + oh[1] * vs[1] + oh[2] * vs[2]
        possum = (sum(jnp.sum(cur * vs[p], axis=0, keepdims=True)
                      for p in range(_P))
                  - jnp.sum(cur * cur, axis=0, keepdims=True))
        pos_item = jnp.exp(possum / _TAU)
        mat = lax.dot_general(cur, cur, dnT, preferred_element_type=f32)
        me = jnp.exp(mat / _TAU)
        valid_i = (ii < cnt_b).astype(f32)
        negsum = jnp.sum(me * valid_i, axis=0, keepdims=True)
        diag = jnp.sum(me * eye, axis=0, keepdims=True)
        neg = negsum - diag
        per = -jnp.log(pos_item / (pos_item + neg + 1e-8))
        validj = (iota_row < cnt_b).astype(f32)
        acc = acc + (jnp.sum(per * validj, keepdims=True).reshape(1, 1)
                     / cnt_b.astype(f32))
    out_ref[...] = acc / float(_B)


def _run_loss(idxp, gathered, cnt):
    return pl.pallas_call(
        _loss_kernel,
        in_specs=[
            pl.BlockSpec(memory_space=pltpu.SMEM),
            pl.BlockSpec(memory_space=pltpu.VMEM),
            pl.BlockSpec(memory_space=pltpu.VMEM),
        ],
        out_specs=pl.BlockSpec(memory_space=pltpu.VMEM),
        out_shape=jax.ShapeDtypeStruct((1, 1), jnp.float32),
    )(idxp, gathered.reshape(_B, _NQ, _P, _D, _SL), cnt)


def kernel(pred, proj_list, idx, pseudo_label, mask, sample_num):
    del pseudo_label
    pred_t = jnp.transpose(pred.reshape(_B, 4, _N), (1, 0, 2))
    mask2 = mask.reshape(_B, _N).astype(jnp.int32)
    sn = jnp.asarray(sample_num, jnp.int32).reshape(1)
    idxp = jnp.asarray(idx, jnp.int32).reshape(1)
    idx_sel, cnt = _run_select(pred_t, mask2, sn)
    proj4 = proj_list.reshape(_P * _B * _D * _N)
    gathered = _run_gather(proj4, idx_sel)
    out = _run_loss(idxp, gathered, cnt)
    return out.reshape(())
